# baseline (speedup 1.0000x reference)
import functools

import jax
import jax.numpy as jnp
from jax import lax
from jax.experimental import pallas as pl
from jax.experimental.pallas import tpu as pltpu
from jax.experimental.pallas import tpu_sc as plsc

_NUM_CLASSES = 1000
_BATCH = 16384
_NC = 2
_NS = 16
_NW = _NC * _NS
_L = 16
_COLS_PER_W = _BATCH // _NW
_CHUNK = 128
_NCHUNK = _COLS_PER_W // _CHUNK
_SPLIT = 504


@functools.partial(
    pl.kernel,
    out_type=jax.ShapeDtypeStruct((_NUM_CLASSES, _BATCH), jnp.float32),
    mesh=plsc.VectorSubcoreMesh(core_axis_name="c", subcore_axis_name="s"),
    scratch_types=[
        pltpu.VMEM((_NUM_CLASSES, _CHUNK), jnp.float32),
        pltpu.VMEM((_COLS_PER_W,), jnp.int32),
        pltpu.SemaphoreType.DMA,
        pltpu.SemaphoreType.DMA,
        pltpu.SemaphoreType.DMA,
    ],
    compiler_params=pltpu.CompilerParams(
        use_tc_tiling_on_sc=True,
        needs_layout_passes=False,
    ),
)
def _onehot_sc(x_hbm, out_hbm, buf, idx_v, sem_a, sem_b, sem_c):
    wid = lax.axis_index("s") * _NC + lax.axis_index("c")
    base_col = wid * _COLS_PER_W

    pltpu.sync_copy(x_hbm.at[pl.ds(base_col, _COLS_PER_W)], idx_v)

    zvec = jnp.zeros((_L,), jnp.float32)
    onevec = jnp.ones((_L,), jnp.float32)
    lane_iota = lax.iota(jnp.int32, _L)

    def _zero_rows(lo, hi):
        def _zero_body(r, c):
            for k in range(4):
                for j in range(_CHUNK // _L):
                    buf[r * 4 + k, pl.ds(j * _L, _L)] = zvec
            return c

        lax.fori_loop(lo // 4, hi // 4, _zero_body, 0)

    def _positions(c):
        out = []
        for j in range(_CHUNK // _L):
            rows = idx_v[pl.ds(c * _CHUNK + j * _L, _L)]
            cols = j * _L + lane_iota
            out.append((rows, cols))
        return out

    def _dma(c, lo, hi, sem):
        return pltpu.make_async_copy(
            buf.at[pl.ds(lo, hi - lo)],
            out_hbm.at[pl.ds(lo, hi - lo), pl.ds(base_col + c * _CHUNK, _CHUNK)],
            sem,
        )

    _p0 = 248
    pieces = [(0, _p0, sem_a), (_p0, _SPLIT, sem_b), (_SPLIT, _NUM_CLASSES, sem_c)]
    for lo, hi, sem in pieces:
        _zero_rows(lo, hi)
        for rows, cols in _positions(0):
            plsc.store_scatter(
                buf, [rows, cols], onevec, mask=(rows >= lo) & (rows < hi)
            )
        _dma(0, lo, hi, sem).start()

    quarters = [(0, 248, sem_a), (248, 504, sem_b), (504, 752, sem_c),
                (752, _NUM_CLASSES, sem_a)]

    for c in range(_NCHUNK):
        if c == 0:
            for lo, hi, sem in pieces:
                _dma(0, lo, hi, sem).wait()
        else:
            for lo, hi, sem in quarters:
                _dma(c, lo, hi, sem).wait()
        if c + 1 == _NCHUNK:
            break
        for rows, cols in _positions(c):
            plsc.store_scatter(buf, [rows, cols], zvec)
        for rows, cols in _positions(c + 1):
            plsc.store_scatter(buf, [rows, cols], onevec)
        for lo, hi, sem in quarters:
            _dma(c + 1, lo, hi, sem).start()


def kernel(x):
    xi = x.astype(jnp.int32)
    return _onehot_sc(xi).T
